# baseline (speedup 1.0000x reference)
import functools

import jax
import jax.numpy as jnp
from jax import lax
from jax.experimental import pallas as pl
from jax.experimental.pallas import tpu as pltpu
from jax.experimental.pallas import tpu_sc as plsc

N = 10000
NP = 10240
E = 320000
F = 128
NC = 2
NS = 16
NW = NC * NS
CH = 120
NCH = 84
EPW = NCH * CH
SIB = 12
NBLK = NCH // SIB
RPT = NP // NS

_mesh = plsc.VectorSubcoreMesh(core_axis_name="c", subcore_axis_name="s")


def _zero_vmem(ref, nvec):
    z = jnp.zeros((16,), jnp.float32)

    def body(i, _):
        ref[pl.ds(i * 16, 16)] = z
        return 0

    lax.fori_loop(0, nvec, body, 0)


@functools.partial(
    pl.kernel,
    out_type=jax.ShapeDtypeStruct((NC, NP), jnp.float32),
    mesh=_mesh,
    scratch_types=[
        pltpu.VMEM((EPW,), jnp.int32),
        pltpu.VMEM((NP,), jnp.float32),
        pltpu.VMEM((RPT,), jnp.float32),
        pltpu.VMEM((RPT,), jnp.float32),
        pltpu.VMEM_SHARED((NS, NP), jnp.float32),
    ],
    compiler_params=pltpu.CompilerParams(needs_layout_passes=False),
)
def _deg_kernel(dst_hbm, deg_out, idx_v, deg_v, sum_v, tmp_v, stage_s):
    cid = lax.axis_index("c")
    sid = lax.axis_index("s")
    wid = cid * NS + sid

    _zero_vmem(deg_v, NP // 16)
    pltpu.sync_copy(dst_hbm.at[wid], idx_v)

    ones = jnp.full((16,), 1.0, jnp.float32)

    def body(g, _):
        iv = idx_v[pl.ds(g * 16, 16)]
        plsc.addupdate_scatter(deg_v, [iv], ones)
        return 0

    lax.fori_loop(0, EPW // 16, body, 0)

    pltpu.sync_copy(deg_v, stage_s.at[sid])
    plsc.subcore_barrier()

    base = sid * RPT
    pltpu.sync_copy(stage_s.at[0, pl.ds(base, RPT)], sum_v)
    for t in range(1, NS):
        pltpu.sync_copy(stage_s.at[t, pl.ds(base, RPT)], tmp_v)

        def add(j, _):
            sl = pl.ds(j * 16, 16)
            sum_v[sl] = sum_v[sl] + tmp_v[sl]
            return 0

        lax.fori_loop(0, RPT // 16, add, 0)
    pltpu.sync_copy(sum_v, deg_out.at[cid, pl.ds(base, RPT)])


@functools.partial(
    pl.kernel,
    out_type=jax.ShapeDtypeStruct((NC, NP, F), jnp.float32),
    mesh=_mesh,
    scratch_types=[
        pltpu.VMEM((2, SIB, CH), jnp.int32),
        pltpu.VMEM((2, SIB, CH), jnp.int32),
        pltpu.VMEM((CH, F), jnp.float32),
        pltpu.VMEM((CH, F), jnp.float32),
        pltpu.VMEM_SHARED((NP, F), jnp.float32),
        pltpu.SemaphoreType.DMA,
        pltpu.SemaphoreType.DMA,
        pltpu.SemaphoreType.DMA,
    ],
)
def _agg_kernel(src_hbm, dst_hbm, feat_hbm, acc_out,
                si_v, di_v, rows_a, rows_b, acc_s, sem_a, sem_b, sem_i):
    cid = lax.axis_index("c")
    sid = lax.axis_index("s")
    wid = cid * NS + sid

    z = jnp.zeros((16,), jnp.float32)

    def zrow(r, _):
        for j in range(F // 16):
            rows_a[r, pl.ds(j * 16, 16)] = z
        return 0

    lax.fori_loop(0, CH, zrow, 0)
    for r in range(RPT // CH):
        pltpu.sync_copy(rows_a, acc_s.at[pl.ds(sid * RPT + r * CH, CH)])
    rem = RPT % CH
    if rem:
        pltpu.sync_copy(rows_a.at[pl.ds(0, rem)],
                        acc_s.at[pl.ds(sid * RPT + (RPT // CH) * CH, rem)])
    plsc.subcore_barrier()

    def stage_start(b):
        pltpu.async_copy(src_hbm.at[wid, b], si_v.at[b % 2], sem_i)
        pltpu.async_copy(dst_hbm.at[wid, b], di_v.at[b % 2], sem_i)

    def stage_wait(b):
        pltpu.make_async_copy(src_hbm.at[wid, b], si_v.at[b % 2], sem_i).wait()
        pltpu.make_async_copy(dst_hbm.at[wid, b], di_v.at[b % 2], sem_i).wait()

    def g_start(b, r, buf, sem):
        pltpu.async_copy(feat_hbm.at[si_v.at[b % 2, r]], buf, sem)

    def g_wait(b, r, buf, sem):
        pltpu.make_async_copy(feat_hbm.at[si_v.at[b % 2, r]], buf, sem).wait()

    def s_add(b, r, buf):
        pltpu.sync_copy(buf, acc_s.at[di_v.at[b % 2, r]], add=True)

    stage_start(0)
    stage_wait(0)
    g_start(0, 0, rows_a, sem_a)
    g_start(0, 1, rows_b, sem_b)

    for b in range(NBLK):
        if b + 1 < NBLK:
            stage_start(b + 1)

        def body(m, _, b=b):
            l0 = 2 * m
            g_wait(b, l0, rows_a, sem_a)
            s_add(b, l0, rows_a)
            g_start(b, l0 + 2, rows_a, sem_a)
            g_wait(b, l0 + 1, rows_b, sem_b)
            s_add(b, l0 + 1, rows_b)
            g_start(b, l0 + 3, rows_b, sem_b)
            return 0

        lax.fori_loop(0, (SIB - 2) // 2, body, 0)

        g_wait(b, SIB - 2, rows_a, sem_a)
        s_add(b, SIB - 2, rows_a)
        if b + 1 < NBLK:
            stage_wait(b + 1)
            g_start(b + 1, 0, rows_a, sem_a)
        g_wait(b, SIB - 1, rows_b, sem_b)
        s_add(b, SIB - 1, rows_b)
        if b + 1 < NBLK:
            g_start(b + 1, 1, rows_b, sem_b)

    plsc.subcore_barrier()
    pltpu.sync_copy(acc_s.at[pl.ds(sid * RPT, RPT)],
                    acc_out.at[cid, pl.ds(sid * RPT, RPT)])


_BR = NP // 8


def _p2_body(degt_ref, x_ref, dis_ref, xs_ref):
    deg = degt_ref[:, 0:1] + degt_ref[:, 1:2] + 1.0
    dis = lax.rsqrt(deg)
    dis_ref[...] = dis
    xs_ref[...] = dis * x_ref[...]


def _p4_body(acc_ref, xs_ref, dis_ref, w1_ref, b1_ref, w2_ref, gs_ref):
    z1 = dis_ref[...] * (acc_ref[0] + acc_ref[1] + xs_ref[...])
    h = jnp.dot(z1, w1_ref[...], preferred_element_type=jnp.float32)
    h = jnp.maximum(h + b1_ref[...], 0.0)
    g = jnp.dot(h, w2_ref[...], preferred_element_type=jnp.float32)
    gs_ref[...] = dis_ref[...] * g


def _p6_body(acc_ref, gs_ref, dis_ref, b2_ref, out_ref):
    out_ref[...] = (dis_ref[...] * (acc_ref[0] + acc_ref[1] + gs_ref[...])
                    + b2_ref[...])


def _rows(i):
    return (i, 0)


def _full(i):
    return (0, 0)


_p2_call = pl.pallas_call(
    _p2_body,
    grid=(8,),
    in_specs=[
        pl.BlockSpec((_BR, 2), _rows),
        pl.BlockSpec((_BR, F), _rows),
    ],
    out_specs=[
        pl.BlockSpec((_BR, 1), _rows),
        pl.BlockSpec((_BR, F), _rows),
    ],
    out_shape=[
        jax.ShapeDtypeStruct((NP, 1), jnp.float32),
        jax.ShapeDtypeStruct((NP, F), jnp.float32),
    ],
)

_p4_call = pl.pallas_call(
    _p4_body,
    grid=(8,),
    in_specs=[
        pl.BlockSpec((NC, _BR, F), lambda i: (0, i, 0)),
        pl.BlockSpec((_BR, F), _rows),
        pl.BlockSpec((_BR, 1), _rows),
        pl.BlockSpec((F, 2 * F), _full),
        pl.BlockSpec((1, 2 * F), _full),
        pl.BlockSpec((2 * F, F), _full),
    ],
    out_specs=pl.BlockSpec((_BR, F), _rows),
    out_shape=jax.ShapeDtypeStruct((NP, F), jnp.float32),
)

_p6_call = pl.pallas_call(
    _p6_body,
    grid=(8,),
    in_specs=[
        pl.BlockSpec((NC, _BR, F), lambda i: (0, i, 0)),
        pl.BlockSpec((_BR, F), _rows),
        pl.BlockSpec((_BR, 1), _rows),
        pl.BlockSpec((1, F), _full),
    ],
    out_specs=pl.BlockSpec((_BR, F), _rows),
    out_shape=jax.ShapeDtypeStruct((NP, F), jnp.float32),
)


def kernel(x, edge_index, W1, b1, W2, b2):
    ei = edge_index.astype(jnp.int32)
    pad = EPW - E // NW
    junk = (N + jnp.arange(NW * pad, dtype=jnp.int32) % (NP - N)).reshape(NW, pad)
    src2 = jnp.pad(ei[0].reshape(NW, E // NW), ((0, 0), (0, pad)))
    dst2 = jnp.concatenate([ei[1].reshape(NW, E // NW), junk], axis=1)
    src3 = src2.reshape(NW, NBLK, SIB, CH)
    dst3 = dst2.reshape(NW, NBLK, SIB, CH)

    xpad = jnp.zeros((NP, F), x.dtype).at[:N].set(x)

    degp = _deg_kernel(dst2)
    dis, xs = _p2_call(degp.T, xpad)
    acc1 = _agg_kernel(src3, dst3, xs)
    gs = _p4_call(acc1, xs, dis, W1, b1.reshape(1, -1), W2)
    acc2 = _agg_kernel(src3, dst3, gs)
    out = _p6_call(acc2, gs, dis, b2.reshape(1, -1))
    return out[:N]


# speedup vs baseline: 1.2743x; 1.2743x over previous
import functools

import jax
import jax.numpy as jnp
from jax import lax
from jax.experimental import pallas as pl
from jax.experimental.pallas import tpu as pltpu
from jax.experimental.pallas import tpu_sc as plsc

N = 10000
NP = 10240
E = 320000
F = 128
NC = 2
NS = 16
NW = NC * NS
CH = 80
NCH = 125
EPW = NCH * CH
SIB = 25
NBLK = NCH // SIB
RPT = NP // NS

_mesh = plsc.VectorSubcoreMesh(core_axis_name="c", subcore_axis_name="s")


def _zero_vmem(ref, nvec):
    z = jnp.zeros((16,), jnp.float32)

    def body(i, _):
        ref[pl.ds(i * 16, 16)] = z
        return 0

    lax.fori_loop(0, nvec, body, 0)


@functools.partial(
    pl.kernel,
    out_type=jax.ShapeDtypeStruct((NC, NP), jnp.float32),
    mesh=_mesh,
    scratch_types=[
        pltpu.VMEM((EPW,), jnp.int32),
        pltpu.VMEM((NP,), jnp.float32),
        pltpu.VMEM((RPT,), jnp.float32),
        pltpu.VMEM((RPT,), jnp.float32),
        pltpu.VMEM_SHARED((NS, NP), jnp.float32),
    ],
    compiler_params=pltpu.CompilerParams(needs_layout_passes=False),
)
def _deg_kernel(dst_hbm, deg_out, idx_v, deg_v, sum_v, tmp_v, stage_s):
    cid = lax.axis_index("c")
    sid = lax.axis_index("s")
    wid = cid * NS + sid

    _zero_vmem(deg_v, NP // 16)
    pltpu.sync_copy(dst_hbm.at[wid], idx_v)

    ones = jnp.full((16,), 1.0, jnp.float32)

    def body(g, _):
        iv = idx_v[pl.ds(g * 16, 16)]
        plsc.addupdate_scatter(deg_v, [iv], ones)
        return 0

    lax.fori_loop(0, EPW // 16, body, 0)

    pltpu.sync_copy(deg_v, stage_s.at[sid])
    plsc.subcore_barrier()

    base = sid * RPT
    pltpu.sync_copy(stage_s.at[0, pl.ds(base, RPT)], sum_v)
    for t in range(1, NS):
        pltpu.sync_copy(stage_s.at[t, pl.ds(base, RPT)], tmp_v)

        def add(j, _):
            sl = pl.ds(j * 16, 16)
            sum_v[sl] = sum_v[sl] + tmp_v[sl]
            return 0

        lax.fori_loop(0, RPT // 16, add, 0)
    pltpu.sync_copy(sum_v, deg_out.at[cid, pl.ds(base, RPT)])


@functools.partial(
    pl.kernel,
    out_type=jax.ShapeDtypeStruct((NC, NP, F), jnp.float32),
    mesh=_mesh,
    scratch_types=[
        pltpu.VMEM((2, SIB, CH), jnp.int32),
        pltpu.VMEM((2, SIB, CH), jnp.int32),
        pltpu.VMEM((CH, F), jnp.float32),
        pltpu.VMEM((CH, F), jnp.float32),
        pltpu.VMEM_SHARED((NP, F), jnp.float32),
        pltpu.SemaphoreType.DMA,
        pltpu.SemaphoreType.DMA,
        pltpu.SemaphoreType.DMA,
        pltpu.SemaphoreType.DMA,
        pltpu.SemaphoreType.DMA,
    ],
)
def _agg_kernel(src_hbm, dst_hbm, feat_hbm, acc_out,
                si_v, di_v, rows_a, rows_b, acc_s,
                sem_a, sem_b, sem_i, ssem_a, ssem_b):
    cid = lax.axis_index("c")
    sid = lax.axis_index("s")
    wid = cid * NS + sid

    z = jnp.zeros((16,), jnp.float32)

    def zrow(r, _):
        for j in range(F // 16):
            rows_a[r, pl.ds(j * 16, 16)] = z
        return 0

    lax.fori_loop(0, CH, zrow, 0)
    for r in range(RPT // CH):
        pltpu.sync_copy(rows_a, acc_s.at[pl.ds(sid * RPT + r * CH, CH)])
    rem = RPT % CH
    if rem:
        pltpu.sync_copy(rows_a.at[pl.ds(0, rem)],
                        acc_s.at[pl.ds(sid * RPT + (RPT // CH) * CH, rem)])
    plsc.subcore_barrier()

    def stage_start(b):
        pltpu.async_copy(src_hbm.at[wid, b], si_v.at[b % 2], sem_i)
        pltpu.async_copy(dst_hbm.at[wid, b], di_v.at[b % 2], sem_i)

    def stage_wait(b):
        pltpu.make_async_copy(src_hbm.at[wid, b], si_v.at[b % 2], sem_i).wait()
        pltpu.make_async_copy(dst_hbm.at[wid, b], di_v.at[b % 2], sem_i).wait()

    def g_start(b, r, buf, sem):
        pltpu.async_copy(feat_hbm.at[si_v.at[b % 2, r]], buf, sem)

    def g_wait(b, r, buf, sem):
        pltpu.make_async_copy(feat_hbm.at[si_v.at[b % 2, r]], buf, sem).wait()

    def s_start(b, r, buf, sem):
        pltpu.async_copy(buf, acc_s.at[di_v.at[b % 2, r]], sem, add=True)

    def s_wait(b, r, buf, sem):
        pltpu.make_async_copy(buf, acc_s.at[di_v.at[b % 2, r]], sem).wait()

    stage_start(0)
    stage_wait(0)
    g_start(0, 0, rows_a, sem_a)
    g_start(0, 1, rows_b, sem_b)

    for b in range(NBLK):
        if b + 1 < NBLK:
            stage_start(b + 1)
        par = b % 2
        be, bo = (rows_a, rows_b) if par == 0 else (rows_b, rows_a)
        ge, go = (sem_a, sem_b) if par == 0 else (sem_b, sem_a)
        se, so = (ssem_a, ssem_b) if par == 0 else (ssem_b, ssem_a)

        def body(m, _, b=b, be=be, bo=bo, ge=ge, go=go, se=se, so=so):
            l0 = 2 * m
            g_wait(b, l0, be, ge)
            s_start(b, l0, be, se)
            g_wait(b, l0 + 1, bo, go)
            s_start(b, l0 + 1, bo, so)
            s_wait(b, l0, be, se)
            g_start(b, l0 + 2, be, ge)
            s_wait(b, l0 + 1, bo, so)
            g_start(b, l0 + 3, bo, go)
            return 0

        lax.fori_loop(0, (SIB - 3) // 2, body, 0)

        g_wait(b, SIB - 3, be, ge)
        s_start(b, SIB - 3, be, se)
        g_wait(b, SIB - 2, bo, go)
        s_start(b, SIB - 2, bo, so)
        s_wait(b, SIB - 3, be, se)
        g_start(b, SIB - 1, be, ge)
        s_wait(b, SIB - 2, bo, so)
        if b + 1 < NBLK:
            stage_wait(b + 1)
            g_start(b + 1, 0, bo, go)
        g_wait(b, SIB - 1, be, ge)
        s_start(b, SIB - 1, be, se)
        s_wait(b, SIB - 1, be, se)
        if b + 1 < NBLK:
            g_start(b + 1, 1, be, ge)

    plsc.subcore_barrier()
    pltpu.sync_copy(acc_s.at[pl.ds(sid * RPT, RPT)],
                    acc_out.at[cid, pl.ds(sid * RPT, RPT)])


_BR = NP // 8


def _p2_body(degt_ref, x_ref, dis_ref, xs_ref):
    deg = degt_ref[:, 0:1] + degt_ref[:, 1:2] + 1.0
    dis = lax.rsqrt(deg)
    dis_ref[...] = dis
    xs_ref[...] = dis * x_ref[...]


def _p4_body(acc_ref, xs_ref, dis_ref, w1_ref, b1_ref, w2_ref, gs_ref):
    z1 = dis_ref[...] * (acc_ref[0] + acc_ref[1] + xs_ref[...])
    h = jnp.dot(z1, w1_ref[...], preferred_element_type=jnp.float32)
    h = jnp.maximum(h + b1_ref[...], 0.0)
    g = jnp.dot(h, w2_ref[...], preferred_element_type=jnp.float32)
    gs_ref[...] = dis_ref[...] * g


def _p6_body(acc_ref, gs_ref, dis_ref, b2_ref, out_ref):
    out_ref[...] = (dis_ref[...] * (acc_ref[0] + acc_ref[1] + gs_ref[...])
                    + b2_ref[...])


def _rows(i):
    return (i, 0)


def _full(i):
    return (0, 0)


_p2_call = pl.pallas_call(
    _p2_body,
    grid=(8,),
    in_specs=[
        pl.BlockSpec((_BR, 2), _rows),
        pl.BlockSpec((_BR, F), _rows),
    ],
    out_specs=[
        pl.BlockSpec((_BR, 1), _rows),
        pl.BlockSpec((_BR, F), _rows),
    ],
    out_shape=[
        jax.ShapeDtypeStruct((NP, 1), jnp.float32),
        jax.ShapeDtypeStruct((NP, F), jnp.float32),
    ],
)

_p4_call = pl.pallas_call(
    _p4_body,
    grid=(8,),
    in_specs=[
        pl.BlockSpec((NC, _BR, F), lambda i: (0, i, 0)),
        pl.BlockSpec((_BR, F), _rows),
        pl.BlockSpec((_BR, 1), _rows),
        pl.BlockSpec((F, 2 * F), _full),
        pl.BlockSpec((1, 2 * F), _full),
        pl.BlockSpec((2 * F, F), _full),
    ],
    out_specs=pl.BlockSpec((_BR, F), _rows),
    out_shape=jax.ShapeDtypeStruct((NP, F), jnp.float32),
)

_p6_call = pl.pallas_call(
    _p6_body,
    grid=(8,),
    in_specs=[
        pl.BlockSpec((NC, _BR, F), lambda i: (0, i, 0)),
        pl.BlockSpec((_BR, F), _rows),
        pl.BlockSpec((_BR, 1), _rows),
        pl.BlockSpec((1, F), _full),
    ],
    out_specs=pl.BlockSpec((_BR, F), _rows),
    out_shape=jax.ShapeDtypeStruct((NP, F), jnp.float32),
)


def kernel(x, edge_index, W1, b1, W2, b2):
    ei = edge_index.astype(jnp.int32)
    pad = EPW - E // NW
    junk = (N + jnp.arange(NW * pad, dtype=jnp.int32) % (NP - N)).reshape(NW, pad)
    src2 = jnp.pad(ei[0].reshape(NW, E // NW), ((0, 0), (0, pad)))
    dst2 = jnp.concatenate([ei[1].reshape(NW, E // NW), junk], axis=1)
    src3 = src2.reshape(NW, NBLK, SIB, CH)
    dst3 = dst2.reshape(NW, NBLK, SIB, CH)

    xpad = jnp.zeros((NP, F), x.dtype).at[:N].set(x)

    degp = _deg_kernel(dst2)
    dis, xs = _p2_call(degp.T, xpad)
    acc1 = _agg_kernel(src3, dst3, xs)
    gs = _p4_call(acc1, xs, dis, W1, b1.reshape(1, -1), W2)
    acc2 = _agg_kernel(src3, dst3, gs)
    out = _p6_call(acc2, gs, dis, b2.reshape(1, -1))
    return out[:N]


# speedup vs baseline: 1.5059x; 1.1818x over previous
import functools

import jax
import jax.numpy as jnp
from jax import lax
from jax.experimental import pallas as pl
from jax.experimental.pallas import tpu as pltpu
from jax.experimental.pallas import tpu_sc as plsc

N = 10000
NP = 10240
E = 320000
F = 128
NC = 2
NS = 16
NW = NC * NS
CH = 80
NCH = 125
EPW = NCH * CH
SIB = 25
NBLK = NCH // SIB
RPT = NP // NS

_mesh = plsc.VectorSubcoreMesh(core_axis_name="c", subcore_axis_name="s")


def _zero_vmem(ref, nvec):
    z = jnp.zeros((16,), jnp.float32)

    def body(i, _):
        ref[pl.ds(i * 16, 16)] = z
        return 0

    lax.fori_loop(0, nvec, body, 0)


@functools.partial(
    pl.kernel,
    out_type=jax.ShapeDtypeStruct((NC, NP), jnp.float32),
    mesh=_mesh,
    scratch_types=[
        pltpu.VMEM((EPW,), jnp.int32),
        pltpu.VMEM((NP,), jnp.float32),
        pltpu.VMEM((RPT,), jnp.float32),
        pltpu.VMEM((RPT,), jnp.float32),
        pltpu.VMEM_SHARED((NS, NP), jnp.float32),
    ],
    compiler_params=pltpu.CompilerParams(needs_layout_passes=False),
)
def _deg_kernel(dst_hbm, deg_out, idx_v, deg_v, sum_v, tmp_v, stage_s):
    cid = lax.axis_index("c")
    sid = lax.axis_index("s")
    wid = cid * NS + sid

    _zero_vmem(deg_v, NP // 16)
    pltpu.sync_copy(dst_hbm.at[wid], idx_v)

    ones = jnp.full((16,), 1.0, jnp.float32)

    def body(g, _):
        iv = idx_v[pl.ds(g * 16, 16)]
        plsc.addupdate_scatter(deg_v, [iv], ones)
        return 0

    lax.fori_loop(0, EPW // 16, body, 0)

    pltpu.sync_copy(deg_v, stage_s.at[sid])
    plsc.subcore_barrier()

    base = sid * RPT
    pltpu.sync_copy(stage_s.at[0, pl.ds(base, RPT)], sum_v)
    for t in range(1, NS):
        pltpu.sync_copy(stage_s.at[t, pl.ds(base, RPT)], tmp_v)

        def add(j, _):
            sl = pl.ds(j * 16, 16)
            sum_v[sl] = sum_v[sl] + tmp_v[sl]
            return 0

        lax.fori_loop(0, RPT // 16, add, 0)
    pltpu.sync_copy(sum_v, deg_out.at[cid, pl.ds(base, RPT)])


@functools.partial(
    pl.kernel,
    out_type=jax.ShapeDtypeStruct((NC, NP, F), jnp.float32),
    mesh=_mesh,
    scratch_types=[
        pltpu.VMEM((2, SIB, CH), jnp.int32),
        pltpu.VMEM((2, SIB, CH), jnp.int32),
        pltpu.VMEM((CH, F), jnp.float32),
        pltpu.VMEM((CH, F), jnp.float32),
        pltpu.VMEM_SHARED((NP, F), jnp.float32),
        pltpu.SemaphoreType.DMA,
        pltpu.SemaphoreType.DMA,
        pltpu.SemaphoreType.DMA,
    ],
)
def _agg_kernel(src_hbm, dst_hbm, feat_hbm, acc_out,
                si_v, di_v, rows_a, rows_b, acc_s,
                sem_a, sem_b, sem_i):
    cid = lax.axis_index("c")
    sid = lax.axis_index("s")
    wid = cid * NS + sid

    z = jnp.zeros((16,), jnp.float32)

    def zrow(r, _):
        for j in range(F // 16):
            rows_a[r, pl.ds(j * 16, 16)] = z
        return 0

    lax.fori_loop(0, CH, zrow, 0)
    for r in range(RPT // CH):
        pltpu.sync_copy(rows_a, acc_s.at[pl.ds(sid * RPT + r * CH, CH)])
    rem = RPT % CH
    if rem:
        pltpu.sync_copy(rows_a.at[pl.ds(0, rem)],
                        acc_s.at[pl.ds(sid * RPT + (RPT // CH) * CH, rem)])
    plsc.subcore_barrier()

    def stage_start(b):
        pltpu.async_copy(src_hbm.at[wid, b], si_v.at[b % 2], sem_i)
        pltpu.async_copy(dst_hbm.at[wid, b], di_v.at[b % 2], sem_i)

    def stage_wait(b):
        pltpu.make_async_copy(src_hbm.at[wid, b], si_v.at[b % 2], sem_i).wait()
        pltpu.make_async_copy(dst_hbm.at[wid, b], di_v.at[b % 2], sem_i).wait()

    def g_start(b, r, buf, sem):
        pltpu.async_copy(feat_hbm.at[si_v.at[b % 2, r]], buf, sem)

    def g_wait(b, r, buf, sem):
        pltpu.make_async_copy(feat_hbm.at[si_v.at[b % 2, r]], buf, sem).wait()

    def s_add(b, r, buf):
        pltpu.sync_copy(buf, acc_s.at[di_v.at[b % 2, r]], add=True)

    stage_start(0)
    stage_wait(0)
    g_start(0, 0, rows_a, sem_a)
    g_start(0, 1, rows_b, sem_b)

    for b in range(NBLK):
        if b + 1 < NBLK:
            stage_start(b + 1)
        par = b % 2
        be, bo = (rows_a, rows_b) if par == 0 else (rows_b, rows_a)
        ge, go = (sem_a, sem_b) if par == 0 else (sem_b, sem_a)

        def body(m, _, b=b, be=be, bo=bo, ge=ge, go=go):
            l0 = 2 * m
            g_wait(b, l0, be, ge)
            s_add(b, l0, be)
            g_start(b, l0 + 2, be, ge)
            g_wait(b, l0 + 1, bo, go)
            s_add(b, l0 + 1, bo)
            g_start(b, l0 + 3, bo, go)
            return 0

        lax.fori_loop(0, (SIB - 3) // 2, body, 0)

        g_wait(b, SIB - 3, be, ge)
        s_add(b, SIB - 3, be)
        g_start(b, SIB - 1, be, ge)
        g_wait(b, SIB - 2, bo, go)
        s_add(b, SIB - 2, bo)
        if b + 1 < NBLK:
            stage_wait(b + 1)
            g_start(b + 1, 0, bo, go)
        g_wait(b, SIB - 1, be, ge)
        s_add(b, SIB - 1, be)
        if b + 1 < NBLK:
            g_start(b + 1, 1, be, ge)

    plsc.subcore_barrier()
    pltpu.sync_copy(acc_s.at[pl.ds(sid * RPT, RPT)],
                    acc_out.at[cid, pl.ds(sid * RPT, RPT)])


_BR = NP // 8


def _p2_body(degt_ref, x_ref, dis_ref, xs_ref):
    deg = degt_ref[:, 0:1] + degt_ref[:, 1:2] + 1.0
    dis = lax.rsqrt(deg)
    dis_ref[...] = dis
    xs_ref[...] = dis * x_ref[...]


def _p4_body(acc_ref, xs_ref, dis_ref, w1_ref, b1_ref, w2_ref, gs_ref):
    z1 = dis_ref[...] * (acc_ref[0] + acc_ref[1] + xs_ref[...])
    h = jnp.dot(z1, w1_ref[...], preferred_element_type=jnp.float32)
    h = jnp.maximum(h + b1_ref[...], 0.0)
    g = jnp.dot(h, w2_ref[...], preferred_element_type=jnp.float32)
    gs_ref[...] = dis_ref[...] * g


def _p6_body(acc_ref, gs_ref, dis_ref, b2_ref, out_ref):
    out_ref[...] = (dis_ref[...] * (acc_ref[0] + acc_ref[1] + gs_ref[...])
                    + b2_ref[...])


def _rows(i):
    return (i, 0)


def _full(i):
    return (0, 0)


_p2_call = pl.pallas_call(
    _p2_body,
    grid=(8,),
    in_specs=[
        pl.BlockSpec((_BR, 2), _rows),
        pl.BlockSpec((_BR, F), _rows),
    ],
    out_specs=[
        pl.BlockSpec((_BR, 1), _rows),
        pl.BlockSpec((_BR, F), _rows),
    ],
    out_shape=[
        jax.ShapeDtypeStruct((NP, 1), jnp.float32),
        jax.ShapeDtypeStruct((NP, F), jnp.float32),
    ],
)

_p4_call = pl.pallas_call(
    _p4_body,
    grid=(8,),
    in_specs=[
        pl.BlockSpec((NC, _BR, F), lambda i: (0, i, 0)),
        pl.BlockSpec((_BR, F), _rows),
        pl.BlockSpec((_BR, 1), _rows),
        pl.BlockSpec((F, 2 * F), _full),
        pl.BlockSpec((1, 2 * F), _full),
        pl.BlockSpec((2 * F, F), _full),
    ],
    out_specs=pl.BlockSpec((_BR, F), _rows),
    out_shape=jax.ShapeDtypeStruct((NP, F), jnp.float32),
)

_p6_call = pl.pallas_call(
    _p6_body,
    grid=(8,),
    in_specs=[
        pl.BlockSpec((NC, _BR, F), lambda i: (0, i, 0)),
        pl.BlockSpec((_BR, F), _rows),
        pl.BlockSpec((_BR, 1), _rows),
        pl.BlockSpec((1, F), _full),
    ],
    out_specs=pl.BlockSpec((_BR, F), _rows),
    out_shape=jax.ShapeDtypeStruct((NP, F), jnp.float32),
)


def kernel(x, edge_index, W1, b1, W2, b2):
    ei = edge_index.astype(jnp.int32)
    pad = EPW - E // NW
    junk = (N + jnp.arange(NW * pad, dtype=jnp.int32) % (NP - N)).reshape(NW, pad)
    src2 = jnp.pad(ei[0].reshape(NW, E // NW), ((0, 0), (0, pad)))
    dst2 = jnp.concatenate([ei[1].reshape(NW, E // NW), junk], axis=1)
    src3 = src2.reshape(NW, NBLK, SIB, CH)
    dst3 = dst2.reshape(NW, NBLK, SIB, CH)

    xpad = jnp.zeros((NP, F), x.dtype).at[:N].set(x)

    degp = _deg_kernel(dst2)
    dis, xs = _p2_call(degp.T, xpad)
    acc1 = _agg_kernel(src3, dst3, xs)
    gs = _p4_call(acc1, xs, dis, W1, b1.reshape(1, -1), W2)
    acc2 = _agg_kernel(src3, dst3, gs)
    out = _p6_call(acc2, gs, dis, b2.reshape(1, -1))
    return out[:N]


# speedup vs baseline: 1.5076x; 1.0011x over previous
import functools

import jax
import jax.numpy as jnp
from jax import lax
from jax.experimental import pallas as pl
from jax.experimental.pallas import tpu as pltpu
from jax.experimental.pallas import tpu_sc as plsc

N = 10000
NP = 10240
E = 320000
F = 128
NC = 2
NS = 16
NW = NC * NS
CH = 80
NCH = 125
EPW = NCH * CH
SIB = 25
NBLK = NCH // SIB
RPT = NP // NS

_mesh = plsc.VectorSubcoreMesh(core_axis_name="c", subcore_axis_name="s")


def _zero_vmem(ref, nvec):
    z = jnp.zeros((16,), jnp.float32)

    def body(i, _):
        ref[pl.ds(i * 16, 16)] = z
        return 0

    lax.fori_loop(0, nvec, body, 0)


@functools.partial(
    pl.kernel,
    out_type=jax.ShapeDtypeStruct((NC, NP), jnp.float32),
    mesh=_mesh,
    scratch_types=[
        pltpu.VMEM((EPW,), jnp.int32),
        pltpu.VMEM((NP,), jnp.float32),
        pltpu.VMEM((RPT,), jnp.float32),
        pltpu.VMEM((RPT,), jnp.float32),
        pltpu.VMEM_SHARED((NS, NP), jnp.float32),
    ],
    compiler_params=pltpu.CompilerParams(needs_layout_passes=False),
)
def _deg_kernel(dst_hbm, deg_out, idx_v, deg_v, sum_v, tmp_v, stage_s):
    cid = lax.axis_index("c")
    sid = lax.axis_index("s")
    wid = cid * NS + sid

    _zero_vmem(deg_v, NP // 16)
    pltpu.sync_copy(dst_hbm.at[wid], idx_v)

    ones = jnp.full((16,), 1.0, jnp.float32)

    def body(g, _):
        iv = idx_v[pl.ds(g * 16, 16)]
        plsc.addupdate_scatter(deg_v, [iv], ones)
        return 0

    lax.fori_loop(0, EPW // 16, body, 0)

    pltpu.sync_copy(deg_v, stage_s.at[sid])
    plsc.subcore_barrier()

    base = sid * RPT
    pltpu.sync_copy(stage_s.at[0, pl.ds(base, RPT)], sum_v)
    for t in range(1, NS):
        pltpu.sync_copy(stage_s.at[t, pl.ds(base, RPT)], tmp_v)

        def add(j, _):
            sl = pl.ds(j * 16, 16)
            sum_v[sl] = sum_v[sl] + tmp_v[sl]
            return 0

        lax.fori_loop(0, RPT // 16, add, 0)
    pltpu.sync_copy(sum_v, deg_out.at[cid, pl.ds(base, RPT)])


@functools.partial(
    pl.kernel,
    out_type=jax.ShapeDtypeStruct((NC, NP, F), jnp.float32),
    mesh=_mesh,
    scratch_types=[
        pltpu.VMEM((2, SIB, CH), jnp.int32),
        pltpu.VMEM((2, SIB, CH), jnp.int32),
        pltpu.VMEM((CH, F), jnp.float32),
        pltpu.VMEM((CH, F), jnp.float32),
        pltpu.VMEM_SHARED((NP, F), jnp.float32),
        pltpu.SemaphoreType.DMA,
        pltpu.SemaphoreType.DMA,
        pltpu.SemaphoreType.DMA,
    ],
)
def _agg_kernel(src_hbm, dst_hbm, feat_hbm, acc_out,
                si_v, di_v, rows_a, rows_b, acc_s,
                sem_a, sem_b, sem_i):
    cid = lax.axis_index("c")
    sid = lax.axis_index("s")
    wid = cid * NS + sid

    z = jnp.zeros((16,), jnp.float32)

    def zrow(r, _):
        for j in range(F // 16):
            rows_a[r, pl.ds(j * 16, 16)] = z
        return 0

    lax.fori_loop(0, CH, zrow, 0)
    for r in range(RPT // CH):
        pltpu.sync_copy(rows_a, acc_s.at[pl.ds(sid * RPT + r * CH, CH)])
    plsc.subcore_barrier()

    def stage_start(b):
        pltpu.async_copy(src_hbm.at[wid, b], si_v.at[b % 2], sem_i)
        pltpu.async_copy(dst_hbm.at[wid, b], di_v.at[b % 2], sem_i)

    def stage_wait(b):
        pltpu.make_async_copy(src_hbm.at[wid, b], si_v.at[b % 2], sem_i).wait()
        pltpu.make_async_copy(dst_hbm.at[wid, b], di_v.at[b % 2], sem_i).wait()

    def g_start(b, r, buf, sem):
        pltpu.async_copy(feat_hbm.at[si_v.at[b % 2, r]], buf, sem)

    def g_wait(b, r, buf, sem):
        pltpu.make_async_copy(feat_hbm.at[si_v.at[b % 2, r]], buf, sem).wait()

    def s_add(b, r, buf):
        pltpu.sync_copy(buf, acc_s.at[di_v.at[b % 2, r]], add=True)

    stage_start(0)
    stage_wait(0)
    g_start(0, 0, rows_a, sem_a)
    g_start(0, 1, rows_b, sem_b)

    for b in range(NBLK):
        if b + 1 < NBLK:
            stage_start(b + 1)
        par = b % 2
        be, bo = (rows_a, rows_b) if par == 0 else (rows_b, rows_a)
        ge, go = (sem_a, sem_b) if par == 0 else (sem_b, sem_a)

        def body(m, _, b=b, be=be, bo=bo, ge=ge, go=go):
            l0 = 2 * m
            g_wait(b, l0, be, ge)
            s_add(b, l0, be)
            g_start(b, l0 + 2, be, ge)
            g_wait(b, l0 + 1, bo, go)
            s_add(b, l0 + 1, bo)
            g_start(b, l0 + 3, bo, go)
            return 0

        lax.fori_loop(0, (SIB - 3) // 2, body, 0)

        g_wait(b, SIB - 3, be, ge)
        s_add(b, SIB - 3, be)
        g_start(b, SIB - 1, be, ge)
        g_wait(b, SIB - 2, bo, go)
        s_add(b, SIB - 2, bo)
        if b + 1 < NBLK:
            stage_wait(b + 1)
            g_start(b + 1, 0, bo, go)
        g_wait(b, SIB - 1, be, ge)
        s_add(b, SIB - 1, be)
        if b + 1 < NBLK:
            g_start(b + 1, 1, be, ge)

    plsc.subcore_barrier()
    pltpu.sync_copy(acc_s.at[pl.ds(sid * RPT, RPT)],
                    acc_out.at[cid, pl.ds(sid * RPT, RPT)])


_GRID = 10
_BR = N // _GRID


def _p2_body(degt_ref, x_ref, dis_ref, xs_ref):
    deg = degt_ref[:, 0:1] + degt_ref[:, 1:2] + 1.0
    dis = lax.rsqrt(deg)
    dis_ref[...] = dis
    xs_ref[...] = dis * x_ref[...]


def _p4_body(acc_ref, xs_ref, dis_ref, w1_ref, b1_ref, w2_ref, gs_ref):
    z1 = dis_ref[...] * (acc_ref[0] + acc_ref[1] + xs_ref[...])
    h = jnp.dot(z1, w1_ref[...], preferred_element_type=jnp.float32)
    h = jnp.maximum(h + b1_ref[...], 0.0)
    g = jnp.dot(h, w2_ref[...], preferred_element_type=jnp.float32)
    gs_ref[...] = dis_ref[...] * g


def _p6_body(acc_ref, gs_ref, dis_ref, b2_ref, out_ref):
    out_ref[...] = (dis_ref[...] * (acc_ref[0] + acc_ref[1] + gs_ref[...])
                    + b2_ref[...])


def _rows(i):
    return (i, 0)


def _full(i):
    return (0, 0)


_p2_call = pl.pallas_call(
    _p2_body,
    grid=(_GRID,),
    in_specs=[
        pl.BlockSpec((_BR, 2), _rows),
        pl.BlockSpec((_BR, F), _rows),
    ],
    out_specs=[
        pl.BlockSpec((_BR, 1), _rows),
        pl.BlockSpec((_BR, F), _rows),
    ],
    out_shape=[
        jax.ShapeDtypeStruct((N, 1), jnp.float32),
        jax.ShapeDtypeStruct((N, F), jnp.float32),
    ],
)

_p4_call = pl.pallas_call(
    _p4_body,
    grid=(_GRID,),
    in_specs=[
        pl.BlockSpec((NC, _BR, F), lambda i: (0, i, 0)),
        pl.BlockSpec((_BR, F), _rows),
        pl.BlockSpec((_BR, 1), _rows),
        pl.BlockSpec((F, 2 * F), _full),
        pl.BlockSpec((1, 2 * F), _full),
        pl.BlockSpec((2 * F, F), _full),
    ],
    out_specs=pl.BlockSpec((_BR, F), _rows),
    out_shape=jax.ShapeDtypeStruct((N, F), jnp.float32),
)

_p6_call = pl.pallas_call(
    _p6_body,
    grid=(_GRID,),
    in_specs=[
        pl.BlockSpec((NC, _BR, F), lambda i: (0, i, 0)),
        pl.BlockSpec((_BR, F), _rows),
        pl.BlockSpec((_BR, 1), _rows),
        pl.BlockSpec((1, F), _full),
    ],
    out_specs=pl.BlockSpec((_BR, F), _rows),
    out_shape=jax.ShapeDtypeStruct((N, F), jnp.float32),
)


def kernel(x, edge_index, W1, b1, W2, b2):
    ei = edge_index.astype(jnp.int32)
    src3 = ei[0].reshape(NW, NBLK, SIB, CH)
    dst3 = ei[1].reshape(NW, NBLK, SIB, CH)
    dst2 = ei[1].reshape(NW, EPW)

    degp = _deg_kernel(dst2)
    degt = degp.T[:N]
    dis, xs = _p2_call(degt, x)
    acc1 = _agg_kernel(src3, dst3, xs)
    gs = _p4_call(acc1, xs, dis, W1, b1.reshape(1, -1), W2)
    acc2 = _agg_kernel(src3, dst3, gs)
    return _p6_call(acc2, gs, dis, b2.reshape(1, -1))


# speedup vs baseline: 1.6893x; 1.1206x over previous
import functools

import jax
import jax.numpy as jnp
from jax import lax
from jax.experimental import pallas as pl
from jax.experimental.pallas import tpu as pltpu
from jax.experimental.pallas import tpu_sc as plsc

N = 10000
NP = 10240
E = 320000
F = 128
NC = 2
NS = 16
NW = NC * NS
CH = 40
NCH = 250
EPW = NCH * CH
SIB = 50
NBLK = NCH // SIB
ND = 4
RPT = NP // NS

_mesh = plsc.VectorSubcoreMesh(core_axis_name="c", subcore_axis_name="s")


def _zero_vmem(ref, nvec):
    z = jnp.zeros((16,), jnp.float32)

    def body(i, _):
        ref[pl.ds(i * 16, 16)] = z
        return 0

    lax.fori_loop(0, nvec, body, 0)


@functools.partial(
    pl.kernel,
    out_type=jax.ShapeDtypeStruct((NC, NP), jnp.float32),
    mesh=_mesh,
    scratch_types=[
        pltpu.VMEM((EPW,), jnp.int32),
        pltpu.VMEM((NP,), jnp.float32),
        pltpu.VMEM((RPT,), jnp.float32),
        pltpu.VMEM((RPT,), jnp.float32),
        pltpu.VMEM_SHARED((NS, NP), jnp.float32),
    ],
    compiler_params=pltpu.CompilerParams(needs_layout_passes=False),
)
def _deg_kernel(dst_hbm, deg_out, idx_v, deg_v, sum_v, tmp_v, stage_s):
    cid = lax.axis_index("c")
    sid = lax.axis_index("s")
    wid = cid * NS + sid

    _zero_vmem(deg_v, NP // 16)
    pltpu.sync_copy(dst_hbm.at[wid], idx_v)

    ones = jnp.full((16,), 1.0, jnp.float32)

    def body(g, _):
        iv = idx_v[pl.ds(g * 16, 16)]
        plsc.addupdate_scatter(deg_v, [iv], ones)
        return 0

    lax.fori_loop(0, EPW // 16, body, 0)

    pltpu.sync_copy(deg_v, stage_s.at[sid])
    plsc.subcore_barrier()

    base = sid * RPT
    pltpu.sync_copy(stage_s.at[0, pl.ds(base, RPT)], sum_v)
    for t in range(1, NS):
        pltpu.sync_copy(stage_s.at[t, pl.ds(base, RPT)], tmp_v)

        def add(j, _):
            sl = pl.ds(j * 16, 16)
            sum_v[sl] = sum_v[sl] + tmp_v[sl]
            return 0

        lax.fori_loop(0, RPT // 16, add, 0)
    pltpu.sync_copy(sum_v, deg_out.at[cid, pl.ds(base, RPT)])


@functools.partial(
    pl.kernel,
    out_type=jax.ShapeDtypeStruct((NC, NP, F), jnp.float32),
    mesh=_mesh,
    scratch_types=[
        pltpu.VMEM((2, SIB, CH), jnp.int32),
        pltpu.VMEM((2, SIB, CH), jnp.int32),
        pltpu.VMEM((CH, F), jnp.float32),
        pltpu.VMEM((CH, F), jnp.float32),
        pltpu.VMEM((CH, F), jnp.float32),
        pltpu.VMEM((CH, F), jnp.float32),
        pltpu.VMEM_SHARED((NP, F), jnp.float32),
        pltpu.SemaphoreType.DMA,
        pltpu.SemaphoreType.DMA,
        pltpu.SemaphoreType.DMA,
        pltpu.SemaphoreType.DMA,
        pltpu.SemaphoreType.DMA,
    ],
)
def _agg_kernel(src_hbm, dst_hbm, feat_hbm, acc_out,
                si_v, di_v, rows_0, rows_1, rows_2, rows_3, acc_s,
                sem_0, sem_1, sem_2, sem_3, sem_i):
    cid = lax.axis_index("c")
    sid = lax.axis_index("s")
    wid = cid * NS + sid

    z = jnp.zeros((16,), jnp.float32)

    def zrow(r, _):
        for j in range(F // 16):
            rows_0[r, pl.ds(j * 16, 16)] = z
        return 0

    lax.fori_loop(0, CH, zrow, 0)
    for r in range(RPT // CH):
        pltpu.sync_copy(rows_0, acc_s.at[pl.ds(sid * RPT + r * CH, CH)])
    plsc.subcore_barrier()

    def stage_start(b):
        pltpu.async_copy(src_hbm.at[wid, b], si_v.at[b % 2], sem_i)
        pltpu.async_copy(dst_hbm.at[wid, b], di_v.at[b % 2], sem_i)

    def stage_wait(b):
        pltpu.make_async_copy(src_hbm.at[wid, b], si_v.at[b % 2], sem_i).wait()
        pltpu.make_async_copy(dst_hbm.at[wid, b], di_v.at[b % 2], sem_i).wait()

    def g_start(b, r, buf, sem):
        pltpu.async_copy(feat_hbm.at[si_v.at[b % 2, r]], buf, sem)

    def g_wait(b, r, buf, sem):
        pltpu.make_async_copy(feat_hbm.at[si_v.at[b % 2, r]], buf, sem).wait()

    def s_add(b, r, buf):
        pltpu.sync_copy(buf, acc_s.at[di_v.at[b % 2, r]], add=True)

    bufs = (rows_0, rows_1, rows_2, rows_3)
    sems = (sem_0, sem_1, sem_2, sem_3)

    stage_start(0)
    stage_wait(0)
    for j in range(ND):
        g_start(0, j, bufs[j], sems[j])

    for b in range(NBLK):
        if b + 1 < NBLK:
            stage_start(b + 1)
        rot = (SIB * b) % ND
        b4 = tuple(bufs[(rot + j) % ND] for j in range(ND))
        s4 = tuple(sems[(rot + j) % ND] for j in range(ND))

        def body(m, _, b=b, b4=b4, s4=s4):
            l0 = ND * m
            for j in range(ND):
                g_wait(b, l0 + j, b4[j], s4[j])
                s_add(b, l0 + j, b4[j])
                g_start(b, l0 + j + ND, b4[j], s4[j])
            return 0

        nfull = (SIB - ND - (ND - 1) - 1) // ND + 1
        lax.fori_loop(0, nfull, body, 0)

        for l in range(ND * nfull, SIB):
            bf = b4[l % ND]
            sm = s4[l % ND]
            g_wait(b, l, bf, sm)
            s_add(b, l, bf)
            nxt = l + ND
            if nxt < SIB:
                g_start(b, nxt, bf, sm)
            elif b + 1 < NBLK:
                if nxt == SIB:
                    stage_wait(b + 1)
                g_start(b + 1, nxt - SIB, bf, sm)

    plsc.subcore_barrier()
    pltpu.sync_copy(acc_s.at[pl.ds(sid * RPT, RPT)],
                    acc_out.at[cid, pl.ds(sid * RPT, RPT)])


_GRID = 10
_BR = N // _GRID


def _p2_body(degt_ref, x_ref, dis_ref, xs_ref):
    deg = degt_ref[:, 0:1] + degt_ref[:, 1:2] + 1.0
    dis = lax.rsqrt(deg)
    dis_ref[...] = dis
    xs_ref[...] = dis * x_ref[...]


def _p4_body(acc_ref, xs_ref, dis_ref, w1_ref, b1_ref, w2_ref, gs_ref):
    z1 = dis_ref[...] * (acc_ref[0] + acc_ref[1] + xs_ref[...])
    h = jnp.dot(z1, w1_ref[...], preferred_element_type=jnp.float32)
    h = jnp.maximum(h + b1_ref[...], 0.0)
    g = jnp.dot(h, w2_ref[...], preferred_element_type=jnp.float32)
    gs_ref[...] = dis_ref[...] * g


def _p6_body(acc_ref, gs_ref, dis_ref, b2_ref, out_ref):
    out_ref[...] = (dis_ref[...] * (acc_ref[0] + acc_ref[1] + gs_ref[...])
                    + b2_ref[...])


def _rows(i):
    return (i, 0)


def _full(i):
    return (0, 0)


_p2_call = pl.pallas_call(
    _p2_body,
    grid=(_GRID,),
    in_specs=[
        pl.BlockSpec((_BR, 2), _rows),
        pl.BlockSpec((_BR, F), _rows),
    ],
    out_specs=[
        pl.BlockSpec((_BR, 1), _rows),
        pl.BlockSpec((_BR, F), _rows),
    ],
    out_shape=[
        jax.ShapeDtypeStruct((N, 1), jnp.float32),
        jax.ShapeDtypeStruct((N, F), jnp.float32),
    ],
)

_p4_call = pl.pallas_call(
    _p4_body,
    grid=(_GRID,),
    in_specs=[
        pl.BlockSpec((NC, _BR, F), lambda i: (0, i, 0)),
        pl.BlockSpec((_BR, F), _rows),
        pl.BlockSpec((_BR, 1), _rows),
        pl.BlockSpec((F, 2 * F), _full),
        pl.BlockSpec((1, 2 * F), _full),
        pl.BlockSpec((2 * F, F), _full),
    ],
    out_specs=pl.BlockSpec((_BR, F), _rows),
    out_shape=jax.ShapeDtypeStruct((N, F), jnp.float32),
)

_p6_call = pl.pallas_call(
    _p6_body,
    grid=(_GRID,),
    in_specs=[
        pl.BlockSpec((NC, _BR, F), lambda i: (0, i, 0)),
        pl.BlockSpec((_BR, F), _rows),
        pl.BlockSpec((_BR, 1), _rows),
        pl.BlockSpec((1, F), _full),
    ],
    out_specs=pl.BlockSpec((_BR, F), _rows),
    out_shape=jax.ShapeDtypeStruct((N, F), jnp.float32),
)


def kernel(x, edge_index, W1, b1, W2, b2):
    ei = edge_index.astype(jnp.int32)
    src3 = ei[0].reshape(NW, NBLK, SIB, CH)
    dst3 = ei[1].reshape(NW, NBLK, SIB, CH)
    dst2 = ei[1].reshape(NW, EPW)

    degp = _deg_kernel(dst2)
    degt = degp.T[:N]
    dis, xs = _p2_call(degt, x)
    acc1 = _agg_kernel(src3, dst3, xs)
    gs = _p4_call(acc1, xs, dis, W1, b1.reshape(1, -1), W2)
    acc2 = _agg_kernel(src3, dst3, gs)
    return _p6_call(acc2, gs, dis, b2.reshape(1, -1))


# speedup vs baseline: 1.6981x; 1.0052x over previous
import functools

import jax
import jax.numpy as jnp
from jax import lax
from jax.experimental import pallas as pl
from jax.experimental.pallas import tpu as pltpu
from jax.experimental.pallas import tpu_sc as plsc

N = 10000
NP = 10240
E = 320000
F = 128
NC = 2
NS = 16
NW = NC * NS
CH = 40
NCH = 250
EPW = NCH * CH
SIB = 50
NBLK = NCH // SIB
ND = 4
RPT = NP // NS

_mesh = plsc.VectorSubcoreMesh(core_axis_name="c", subcore_axis_name="s")


def _zero_vmem(ref, nvec):
    z = jnp.zeros((16,), jnp.float32)

    def body(i, _):
        ref[pl.ds(i * 16, 16)] = z
        return 0

    lax.fori_loop(0, nvec, body, 0)


@functools.partial(
    pl.kernel,
    out_type=jax.ShapeDtypeStruct((NC, NP), jnp.float32),
    mesh=_mesh,
    scratch_types=[
        pltpu.VMEM((EPW,), jnp.int32),
        pltpu.VMEM((NP,), jnp.float32),
        pltpu.VMEM((RPT,), jnp.float32),
        pltpu.VMEM((RPT,), jnp.float32),
        pltpu.VMEM_SHARED((NS, NP), jnp.float32),
    ],
    compiler_params=pltpu.CompilerParams(needs_layout_passes=False),
)
def _deg_kernel(dst_hbm, deg_out, idx_v, deg_v, sum_v, tmp_v, stage_s):
    cid = lax.axis_index("c")
    sid = lax.axis_index("s")
    wid = cid * NS + sid

    _zero_vmem(deg_v, NP // 16)
    pltpu.sync_copy(dst_hbm.at[wid], idx_v)

    ones = jnp.full((16,), 1.0, jnp.float32)

    def body(g, _):
        iv = idx_v[pl.ds(g * 16, 16)]
        plsc.addupdate_scatter(deg_v, [iv], ones)
        return 0

    lax.fori_loop(0, EPW // 16, body, 0)

    pltpu.sync_copy(deg_v, stage_s.at[sid])
    plsc.subcore_barrier()

    base = sid * RPT
    pltpu.sync_copy(stage_s.at[0, pl.ds(base, RPT)], sum_v)
    for t in range(1, NS):
        pltpu.sync_copy(stage_s.at[t, pl.ds(base, RPT)], tmp_v)

        def add(j, _):
            sl = pl.ds(j * 16, 16)
            sum_v[sl] = sum_v[sl] + tmp_v[sl]
            return 0

        lax.fori_loop(0, RPT // 16, add, 0)
    pltpu.sync_copy(sum_v, deg_out.at[cid, pl.ds(base, RPT)])


@functools.partial(
    pl.kernel,
    out_type=jax.ShapeDtypeStruct((NC, NP, F), jnp.float32),
    mesh=_mesh,
    scratch_types=[
        pltpu.VMEM((2, SIB, CH), jnp.int32),
        pltpu.VMEM((2, SIB, CH), jnp.int32),
        pltpu.VMEM((CH, F), jnp.float32),
        pltpu.VMEM((CH, F), jnp.float32),
        pltpu.VMEM((CH, F), jnp.float32),
        pltpu.VMEM((CH, F), jnp.float32),
        pltpu.VMEM_SHARED((NP, F), jnp.float32),
        pltpu.SemaphoreType.DMA,
        pltpu.SemaphoreType.DMA,
        pltpu.SemaphoreType.DMA,
        pltpu.SemaphoreType.DMA,
        pltpu.SemaphoreType.DMA,
    ],
)
def _agg_kernel(src_hbm, dst_hbm, feat_hbm, acc_out,
                si_v, di_v, rows_0, rows_1, rows_2, rows_3, acc_s,
                sem_0, sem_1, sem_2, sem_3, sem_i):
    cid = lax.axis_index("c")
    sid = lax.axis_index("s")
    wid = cid * NS + sid

    z = jnp.zeros((16,), jnp.float32)

    def zrow(r, _):
        for j in range(F // 16):
            rows_0[r, pl.ds(j * 16, 16)] = z
        return 0

    lax.fori_loop(0, CH, zrow, 0)
    for r in range(RPT // CH):
        pltpu.async_copy(rows_0, acc_s.at[pl.ds(sid * RPT + r * CH, CH)],
                         sem_i)
    for r in range(RPT // CH):
        pltpu.make_async_copy(
            rows_0, acc_s.at[pl.ds(sid * RPT + r * CH, CH)], sem_i).wait()
    plsc.subcore_barrier()

    def stage_start(b):
        pltpu.async_copy(src_hbm.at[wid, b], si_v.at[b % 2], sem_i)
        pltpu.async_copy(dst_hbm.at[wid, b], di_v.at[b % 2], sem_i)

    def stage_wait(b):
        pltpu.make_async_copy(src_hbm.at[wid, b], si_v.at[b % 2], sem_i).wait()
        pltpu.make_async_copy(dst_hbm.at[wid, b], di_v.at[b % 2], sem_i).wait()

    def g_start(b, r, buf, sem):
        pltpu.async_copy(feat_hbm.at[si_v.at[b % 2, r]], buf, sem)

    def g_wait(b, r, buf, sem):
        pltpu.make_async_copy(feat_hbm.at[si_v.at[b % 2, r]], buf, sem).wait()

    def s_add(b, r, buf):
        pltpu.sync_copy(buf, acc_s.at[di_v.at[b % 2, r]], add=True)

    bufs = (rows_0, rows_1, rows_2, rows_3)
    sems = (sem_0, sem_1, sem_2, sem_3)

    stage_start(0)
    stage_wait(0)
    for j in range(ND):
        g_start(0, j, bufs[j], sems[j])

    for b in range(NBLK):
        if b + 1 < NBLK:
            stage_start(b + 1)
        rot = (SIB * b) % ND
        b4 = tuple(bufs[(rot + j) % ND] for j in range(ND))
        s4 = tuple(sems[(rot + j) % ND] for j in range(ND))

        def body(m, _, b=b, b4=b4, s4=s4):
            l0 = ND * m
            for j in range(ND):
                g_wait(b, l0 + j, b4[j], s4[j])
                s_add(b, l0 + j, b4[j])
                g_start(b, l0 + j + ND, b4[j], s4[j])
            return 0

        nfull = (SIB - ND - (ND - 1) - 1) // ND + 1
        lax.fori_loop(0, nfull, body, 0)

        for l in range(ND * nfull, SIB):
            bf = b4[l % ND]
            sm = s4[l % ND]
            g_wait(b, l, bf, sm)
            s_add(b, l, bf)
            nxt = l + ND
            if nxt < SIB:
                g_start(b, nxt, bf, sm)
            elif b + 1 < NBLK:
                if nxt == SIB:
                    stage_wait(b + 1)
                g_start(b + 1, nxt - SIB, bf, sm)

    plsc.subcore_barrier()
    pltpu.sync_copy(acc_s.at[pl.ds(sid * RPT, RPT)],
                    acc_out.at[cid, pl.ds(sid * RPT, RPT)])


_GRID = 10
_BR = N // _GRID


def _p2_body(degt_ref, x_ref, dis_ref, xs_ref):
    deg = degt_ref[:, 0:1] + degt_ref[:, 1:2] + 1.0
    dis = lax.rsqrt(deg)
    dis_ref[...] = dis
    xs_ref[...] = dis * x_ref[...]


def _p4_body(acc_ref, xs_ref, dis_ref, w1_ref, b1_ref, w2_ref, gs_ref):
    z1 = dis_ref[...] * (acc_ref[0] + acc_ref[1] + xs_ref[...])
    h = jnp.dot(z1, w1_ref[...], preferred_element_type=jnp.float32)
    h = jnp.maximum(h + b1_ref[...], 0.0)
    g = jnp.dot(h, w2_ref[...], preferred_element_type=jnp.float32)
    gs_ref[...] = dis_ref[...] * g


def _p6_body(acc_ref, gs_ref, dis_ref, b2_ref, out_ref):
    out_ref[...] = (dis_ref[...] * (acc_ref[0] + acc_ref[1] + gs_ref[...])
                    + b2_ref[...])


def _rows(i):
    return (i, 0)


def _full(i):
    return (0, 0)


_p2_call = pl.pallas_call(
    _p2_body,
    grid=(_GRID,),
    in_specs=[
        pl.BlockSpec((_BR, 2), _rows),
        pl.BlockSpec((_BR, F), _rows),
    ],
    out_specs=[
        pl.BlockSpec((_BR, 1), _rows),
        pl.BlockSpec((_BR, F), _rows),
    ],
    out_shape=[
        jax.ShapeDtypeStruct((N, 1), jnp.float32),
        jax.ShapeDtypeStruct((N, F), jnp.float32),
    ],
)

_p4_call = pl.pallas_call(
    _p4_body,
    grid=(_GRID,),
    in_specs=[
        pl.BlockSpec((NC, _BR, F), lambda i: (0, i, 0)),
        pl.BlockSpec((_BR, F), _rows),
        pl.BlockSpec((_BR, 1), _rows),
        pl.BlockSpec((F, 2 * F), _full),
        pl.BlockSpec((1, 2 * F), _full),
        pl.BlockSpec((2 * F, F), _full),
    ],
    out_specs=pl.BlockSpec((_BR, F), _rows),
    out_shape=jax.ShapeDtypeStruct((N, F), jnp.float32),
)

_p6_call = pl.pallas_call(
    _p6_body,
    grid=(_GRID,),
    in_specs=[
        pl.BlockSpec((NC, _BR, F), lambda i: (0, i, 0)),
        pl.BlockSpec((_BR, F), _rows),
        pl.BlockSpec((_BR, 1), _rows),
        pl.BlockSpec((1, F), _full),
    ],
    out_specs=pl.BlockSpec((_BR, F), _rows),
    out_shape=jax.ShapeDtypeStruct((N, F), jnp.float32),
)


def kernel(x, edge_index, W1, b1, W2, b2):
    ei = edge_index.astype(jnp.int32)
    src3 = ei[0].reshape(NW, NBLK, SIB, CH)
    dst3 = ei[1].reshape(NW, NBLK, SIB, CH)
    dst2 = ei[1].reshape(NW, EPW)

    degp = _deg_kernel(dst2)
    degt = degp.T[:N]
    dis, xs = _p2_call(degt, x)
    acc1 = _agg_kernel(src3, dst3, xs)
    gs = _p4_call(acc1, xs, dis, W1, b1.reshape(1, -1), W2)
    acc2 = _agg_kernel(src3, dst3, gs)
    return _p6_call(acc2, gs, dis, b2.reshape(1, -1))


# speedup vs baseline: 1.7469x; 1.0288x over previous
import functools

import jax
import jax.numpy as jnp
from jax import lax
from jax.experimental import pallas as pl
from jax.experimental.pallas import tpu as pltpu
from jax.experimental.pallas import tpu_sc as plsc

N = 10000
NP = 10240
E = 320000
F = 128
NC = 2
NS = 16
NW = NC * NS
CH = 40
NCH = 250
EPW = NCH * CH
SIB = 50
NBLK = NCH // SIB
ND = 4
RPT = NP // NS

_mesh = plsc.VectorSubcoreMesh(core_axis_name="c", subcore_axis_name="s")


def _zero_vmem(ref, nvec):
    z = jnp.zeros((16,), jnp.float32)

    def body(i, _):
        ref[pl.ds(i * 16, 16)] = z
        return 0

    lax.fori_loop(0, nvec, body, 0)


@functools.partial(
    pl.kernel,
    out_type=jax.ShapeDtypeStruct((NC, NP), jnp.float32),
    mesh=_mesh,
    scratch_types=[
        pltpu.VMEM((EPW,), jnp.int32),
        pltpu.VMEM((NP,), jnp.float32),
        pltpu.VMEM((RPT,), jnp.float32),
        pltpu.VMEM((NS, RPT), jnp.float32),
        pltpu.VMEM_SHARED((NS, NP), jnp.float32),
    ],
    compiler_params=pltpu.CompilerParams(needs_layout_passes=False),
)
def _deg_kernel(dst_hbm, deg_out, idx_v, deg_v, sum_v, tmp_v, stage_s):
    cid = lax.axis_index("c")
    sid = lax.axis_index("s")
    wid = cid * NS + sid

    _zero_vmem(deg_v, NP // 16)
    pltpu.sync_copy(dst_hbm.at[wid], idx_v)

    ones = jnp.full((16,), 1.0, jnp.float32)

    def body(g, _):
        iv = idx_v[pl.ds(g * 16, 16)]
        plsc.addupdate_scatter(deg_v, [iv], ones)
        return 0

    lax.fori_loop(0, EPW // 16, body, 0)

    pltpu.sync_copy(deg_v, stage_s.at[sid])
    plsc.subcore_barrier()

    base = sid * RPT
    pltpu.sync_copy(stage_s.at[:, pl.ds(base, RPT)], tmp_v)

    def add(j, _):
        sl = pl.ds(j * 16, 16)
        acc = tmp_v[0, sl]
        for t in range(1, NS):
            acc = acc + tmp_v[t, sl]
        sum_v[sl] = acc
        return 0

    lax.fori_loop(0, RPT // 16, add, 0)
    pltpu.sync_copy(sum_v, deg_out.at[cid, pl.ds(base, RPT)])


@functools.partial(
    pl.kernel,
    out_type=jax.ShapeDtypeStruct((NC, NP, F), jnp.float32),
    mesh=_mesh,
    scratch_types=[
        pltpu.VMEM((2, SIB, CH), jnp.int32),
        pltpu.VMEM((2, SIB, CH), jnp.int32),
        pltpu.VMEM((CH, F), jnp.float32),
        pltpu.VMEM((CH, F), jnp.float32),
        pltpu.VMEM((CH, F), jnp.float32),
        pltpu.VMEM((CH, F), jnp.float32),
        pltpu.VMEM_SHARED((NP, F), jnp.float32),
        pltpu.SemaphoreType.DMA,
        pltpu.SemaphoreType.DMA,
        pltpu.SemaphoreType.DMA,
        pltpu.SemaphoreType.DMA,
        pltpu.SemaphoreType.DMA,
    ],
)
def _agg_kernel(src_hbm, dst_hbm, feat_hbm, acc_out,
                si_v, di_v, rows_0, rows_1, rows_2, rows_3, acc_s,
                sem_0, sem_1, sem_2, sem_3, sem_i):
    cid = lax.axis_index("c")
    sid = lax.axis_index("s")
    wid = cid * NS + sid

    z = jnp.zeros((16,), jnp.float32)

    def zrow(r, _):
        for j in range(F // 16):
            rows_3[r, pl.ds(j * 16, 16)] = z
        return 0

    lax.fori_loop(0, CH, zrow, 0)
    for r in range(RPT // CH):
        pltpu.async_copy(rows_3, acc_s.at[pl.ds(sid * RPT + r * CH, CH)],
                         sem_3)

    def stage_start(b):
        pltpu.async_copy(src_hbm.at[wid, b], si_v.at[b % 2], sem_i)
        pltpu.async_copy(dst_hbm.at[wid, b], di_v.at[b % 2], sem_i)

    def stage_wait(b):
        pltpu.make_async_copy(src_hbm.at[wid, b], si_v.at[b % 2], sem_i).wait()
        pltpu.make_async_copy(dst_hbm.at[wid, b], di_v.at[b % 2], sem_i).wait()

    def g_start(b, r, buf, sem):
        pltpu.async_copy(feat_hbm.at[si_v.at[b % 2, r]], buf, sem)

    def g_wait(b, r, buf, sem):
        pltpu.make_async_copy(feat_hbm.at[si_v.at[b % 2, r]], buf, sem).wait()

    def s_add(b, r, buf):
        pltpu.sync_copy(buf, acc_s.at[di_v.at[b % 2, r]], add=True)

    bufs = (rows_0, rows_1, rows_2, rows_3)
    sems = (sem_0, sem_1, sem_2, sem_3)

    stage_start(0)
    stage_wait(0)
    for j in range(ND - 1):
        g_start(0, j, bufs[j], sems[j])
    for r in range(RPT // CH):
        pltpu.make_async_copy(
            rows_3, acc_s.at[pl.ds(sid * RPT + r * CH, CH)], sem_3).wait()
    plsc.subcore_barrier()
    g_start(0, ND - 1, bufs[ND - 1], sems[ND - 1])

    for b in range(NBLK):
        if b + 1 < NBLK:
            stage_start(b + 1)
        rot = (SIB * b) % ND
        b4 = tuple(bufs[(rot + j) % ND] for j in range(ND))
        s4 = tuple(sems[(rot + j) % ND] for j in range(ND))

        def body(m, _, b=b, b4=b4, s4=s4):
            l0 = ND * m
            for j in range(ND):
                g_wait(b, l0 + j, b4[j], s4[j])
                s_add(b, l0 + j, b4[j])
                g_start(b, l0 + j + ND, b4[j], s4[j])
            return 0

        nfull = (SIB - ND - (ND - 1) - 1) // ND + 1
        lax.fori_loop(0, nfull, body, 0)

        for l in range(ND * nfull, SIB):
            bf = b4[l % ND]
            sm = s4[l % ND]
            g_wait(b, l, bf, sm)
            s_add(b, l, bf)
            nxt = l + ND
            if nxt < SIB:
                g_start(b, nxt, bf, sm)
            elif b + 1 < NBLK:
                if nxt == SIB:
                    stage_wait(b + 1)
                g_start(b + 1, nxt - SIB, bf, sm)

    plsc.subcore_barrier()
    pltpu.sync_copy(acc_s.at[pl.ds(sid * RPT, RPT)],
                    acc_out.at[cid, pl.ds(sid * RPT, RPT)])


_GRID = 10
_BR = N // _GRID


def _p2_body(degt_ref, x_ref, dis_ref, xs_ref):
    deg = degt_ref[:, 0:1] + degt_ref[:, 1:2] + 1.0
    dis = lax.rsqrt(deg)
    dis_ref[...] = dis
    xs_ref[...] = dis * x_ref[...]


def _p4_body(acc_ref, xs_ref, dis_ref, w1_ref, b1_ref, w2_ref, gs_ref):
    z1 = dis_ref[...] * (acc_ref[0] + acc_ref[1] + xs_ref[...])
    h = jnp.dot(z1, w1_ref[...], preferred_element_type=jnp.float32)
    h = jnp.maximum(h + b1_ref[...], 0.0)
    g = jnp.dot(h, w2_ref[...], preferred_element_type=jnp.float32)
    gs_ref[...] = dis_ref[...] * g


def _p6_body(acc_ref, gs_ref, dis_ref, b2_ref, out_ref):
    out_ref[...] = (dis_ref[...] * (acc_ref[0] + acc_ref[1] + gs_ref[...])
                    + b2_ref[...])


def _rows(i):
    return (i, 0)


def _full(i):
    return (0, 0)


_p2_call = pl.pallas_call(
    _p2_body,
    grid=(_GRID,),
    in_specs=[
        pl.BlockSpec((_BR, 2), _rows),
        pl.BlockSpec((_BR, F), _rows),
    ],
    out_specs=[
        pl.BlockSpec((_BR, 1), _rows),
        pl.BlockSpec((_BR, F), _rows),
    ],
    out_shape=[
        jax.ShapeDtypeStruct((N, 1), jnp.float32),
        jax.ShapeDtypeStruct((N, F), jnp.float32),
    ],
)

_p4_call = pl.pallas_call(
    _p4_body,
    grid=(_GRID,),
    in_specs=[
        pl.BlockSpec((NC, _BR, F), lambda i: (0, i, 0)),
        pl.BlockSpec((_BR, F), _rows),
        pl.BlockSpec((_BR, 1), _rows),
        pl.BlockSpec((F, 2 * F), _full),
        pl.BlockSpec((1, 2 * F), _full),
        pl.BlockSpec((2 * F, F), _full),
    ],
    out_specs=pl.BlockSpec((_BR, F), _rows),
    out_shape=jax.ShapeDtypeStruct((N, F), jnp.float32),
)

_p6_call = pl.pallas_call(
    _p6_body,
    grid=(_GRID,),
    in_specs=[
        pl.BlockSpec((NC, _BR, F), lambda i: (0, i, 0)),
        pl.BlockSpec((_BR, F), _rows),
        pl.BlockSpec((_BR, 1), _rows),
        pl.BlockSpec((1, F), _full),
    ],
    out_specs=pl.BlockSpec((_BR, F), _rows),
    out_shape=jax.ShapeDtypeStruct((N, F), jnp.float32),
)


def kernel(x, edge_index, W1, b1, W2, b2):
    ei = edge_index.astype(jnp.int32)
    src3 = ei[0].reshape(NW, NBLK, SIB, CH)
    dst3 = ei[1].reshape(NW, NBLK, SIB, CH)
    dst2 = ei[1].reshape(NW, EPW)

    degp = _deg_kernel(dst2)
    degt = degp.T[:N]
    dis, xs = _p2_call(degt, x)
    acc1 = _agg_kernel(src3, dst3, xs)
    gs = _p4_call(acc1, xs, dis, W1, b1.reshape(1, -1), W2)
    acc2 = _agg_kernel(src3, dst3, gs)
    return _p6_call(acc2, gs, dis, b2.reshape(1, -1))


# speedup vs baseline: 1.8016x; 1.0313x over previous
import functools

import jax
import jax.numpy as jnp
from jax import lax
from jax.experimental import pallas as pl
from jax.experimental.pallas import tpu as pltpu
from jax.experimental.pallas import tpu_sc as plsc

N = 10000
NP = 10240
E = 320000
F = 128
NC = 2
NS = 16
NW = NC * NS
CH = 40
NCH = 250
EPW = NCH * CH
SIB = 25
NBLK = NCH // SIB
ND = 5
RPT = NP // NS
NPA = 10112
RPA = NPA // NS

_mesh = plsc.VectorSubcoreMesh(core_axis_name="c", subcore_axis_name="s")


def _zero_vmem(ref, nvec):
    z = jnp.zeros((16,), jnp.float32)

    def body(i, _):
        ref[pl.ds(i * 16, 16)] = z
        return 0

    lax.fori_loop(0, nvec, body, 0)


@functools.partial(
    pl.kernel,
    out_type=jax.ShapeDtypeStruct((NC, NP), jnp.float32),
    mesh=_mesh,
    scratch_types=[
        pltpu.VMEM((EPW,), jnp.int32),
        pltpu.VMEM((NP,), jnp.float32),
        pltpu.VMEM((RPT,), jnp.float32),
        pltpu.VMEM((NS, RPT), jnp.float32),
        pltpu.VMEM_SHARED((NS, NP), jnp.float32),
    ],
    compiler_params=pltpu.CompilerParams(needs_layout_passes=False),
)
def _deg_kernel(dst_hbm, deg_out, idx_v, deg_v, sum_v, tmp_v, stage_s):
    cid = lax.axis_index("c")
    sid = lax.axis_index("s")
    wid = cid * NS + sid

    _zero_vmem(deg_v, NP // 16)
    pltpu.sync_copy(dst_hbm.at[wid], idx_v)

    ones = jnp.full((16,), 1.0, jnp.float32)

    def body(g, _):
        iv = idx_v[pl.ds(g * 16, 16)]
        plsc.addupdate_scatter(deg_v, [iv], ones)
        return 0

    lax.fori_loop(0, EPW // 16, body, 0)

    pltpu.sync_copy(deg_v, stage_s.at[sid])
    plsc.subcore_barrier()

    base = sid * RPT
    pltpu.sync_copy(stage_s.at[:, pl.ds(base, RPT)], tmp_v)

    def add(j, _):
        sl = pl.ds(j * 16, 16)
        acc = tmp_v[0, sl]
        for t in range(1, NS):
            acc = acc + tmp_v[t, sl]
        sum_v[sl] = acc
        return 0

    lax.fori_loop(0, RPT // 16, add, 0)
    pltpu.sync_copy(sum_v, deg_out.at[cid, pl.ds(base, RPT)])


@functools.partial(
    pl.kernel,
    out_type=jax.ShapeDtypeStruct((NC, NPA, F), jnp.float32),
    mesh=_mesh,
    scratch_types=[
        pltpu.VMEM((2, SIB, CH), jnp.int32),
        pltpu.VMEM((2, SIB, CH), jnp.int32),
        pltpu.VMEM((CH, F), jnp.float32),
        pltpu.VMEM((CH, F), jnp.float32),
        pltpu.VMEM((CH, F), jnp.float32),
        pltpu.VMEM((CH, F), jnp.float32),
        pltpu.VMEM((CH, F), jnp.float32),
        pltpu.VMEM_SHARED((NPA, F), jnp.float32),
        pltpu.SemaphoreType.DMA,
        pltpu.SemaphoreType.DMA,
        pltpu.SemaphoreType.DMA,
        pltpu.SemaphoreType.DMA,
        pltpu.SemaphoreType.DMA,
        pltpu.SemaphoreType.DMA,
    ],
)
def _agg_kernel(src_hbm, dst_hbm, feat_hbm, acc_out,
                si_v, di_v, rows_0, rows_1, rows_2, rows_3, rows_4, acc_s,
                sem_0, sem_1, sem_2, sem_3, sem_4, sem_i):
    cid = lax.axis_index("c")
    sid = lax.axis_index("s")
    wid = cid * NS + sid

    z = jnp.zeros((16,), jnp.float32)

    def zrow(r, _):
        for j in range(F // 16):
            rows_4[r, pl.ds(j * 16, 16)] = z
        return 0

    lax.fori_loop(0, CH, zrow, 0)

    def init_descs():
        descs = []
        for r in range(RPA // CH):
            descs.append((rows_4, acc_s.at[pl.ds(sid * RPA + r * CH, CH)]))
        rem = RPA % CH
        if rem:
            descs.append((rows_4.at[pl.ds(0, rem)],
                          acc_s.at[pl.ds(sid * RPA + (RPA // CH) * CH, rem)]))
        return descs

    for s, d in init_descs():
        pltpu.async_copy(s, d, sem_4)

    def stage_start(b):
        pltpu.async_copy(src_hbm.at[wid, b], si_v.at[b % 2], sem_i)
        pltpu.async_copy(dst_hbm.at[wid, b], di_v.at[b % 2], sem_i)

    def stage_wait(b):
        pltpu.make_async_copy(src_hbm.at[wid, b], si_v.at[b % 2], sem_i).wait()
        pltpu.make_async_copy(dst_hbm.at[wid, b], di_v.at[b % 2], sem_i).wait()

    def g_start(b, r, buf, sem):
        pltpu.async_copy(feat_hbm.at[si_v.at[b % 2, r]], buf, sem)

    def g_wait(b, r, buf, sem):
        pltpu.make_async_copy(feat_hbm.at[si_v.at[b % 2, r]], buf, sem).wait()

    def s_add(b, r, buf):
        pltpu.sync_copy(buf, acc_s.at[di_v.at[b % 2, r]], add=True)

    bufs = (rows_0, rows_1, rows_2, rows_3, rows_4)
    sems = (sem_0, sem_1, sem_2, sem_3, sem_4)

    stage_start(0)
    stage_wait(0)
    for j in range(ND - 1):
        g_start(0, j, bufs[j], sems[j])
    for s, d in init_descs():
        pltpu.make_async_copy(s, d, sem_4).wait()
    plsc.subcore_barrier()
    g_start(0, ND - 1, bufs[ND - 1], sems[ND - 1])

    for b in range(NBLK):
        if b + 1 < NBLK:
            stage_start(b + 1)
        rot = (SIB * b) % ND
        b4 = tuple(bufs[(rot + j) % ND] for j in range(ND))
        s4 = tuple(sems[(rot + j) % ND] for j in range(ND))

        def body(m, _, b=b, b4=b4, s4=s4):
            l0 = ND * m
            for j in range(ND):
                g_wait(b, l0 + j, b4[j], s4[j])
                s_add(b, l0 + j, b4[j])
                g_start(b, l0 + j + ND, b4[j], s4[j])
            return 0

        nfull = (SIB - ND - (ND - 1) - 1) // ND + 1
        lax.fori_loop(0, nfull, body, 0)

        for l in range(ND * nfull, SIB):
            bf = b4[l % ND]
            sm = s4[l % ND]
            g_wait(b, l, bf, sm)
            s_add(b, l, bf)
            nxt = l + ND
            if nxt < SIB:
                g_start(b, nxt, bf, sm)
            elif b + 1 < NBLK:
                if nxt == SIB:
                    stage_wait(b + 1)
                g_start(b + 1, nxt - SIB, bf, sm)

    plsc.subcore_barrier()
    pltpu.sync_copy(acc_s.at[pl.ds(sid * RPA, RPA)],
                    acc_out.at[cid, pl.ds(sid * RPA, RPA)])


_GRID = 10
_BR = N // _GRID


def _p2_body(degt_ref, x_ref, dis_ref, xs_ref):
    deg = degt_ref[:, 0:1] + degt_ref[:, 1:2] + 1.0
    dis = lax.rsqrt(deg)
    dis_ref[...] = dis
    xs_ref[...] = dis * x_ref[...]


def _p4_body(acc_ref, xs_ref, dis_ref, w1_ref, b1_ref, w2_ref, gs_ref):
    z1 = dis_ref[...] * (acc_ref[0] + acc_ref[1] + xs_ref[...])
    h = jnp.dot(z1, w1_ref[...], preferred_element_type=jnp.float32)
    h = jnp.maximum(h + b1_ref[...], 0.0)
    g = jnp.dot(h, w2_ref[...], preferred_element_type=jnp.float32)
    gs_ref[...] = dis_ref[...] * g


def _p6_body(acc_ref, gs_ref, dis_ref, b2_ref, out_ref):
    out_ref[...] = (dis_ref[...] * (acc_ref[0] + acc_ref[1] + gs_ref[...])
                    + b2_ref[...])


def _rows(i):
    return (i, 0)


def _full(i):
    return (0, 0)


_p2_call = pl.pallas_call(
    _p2_body,
    grid=(_GRID,),
    in_specs=[
        pl.BlockSpec((_BR, 2), _rows),
        pl.BlockSpec((_BR, F), _rows),
    ],
    out_specs=[
        pl.BlockSpec((_BR, 1), _rows),
        pl.BlockSpec((_BR, F), _rows),
    ],
    out_shape=[
        jax.ShapeDtypeStruct((N, 1), jnp.float32),
        jax.ShapeDtypeStruct((N, F), jnp.float32),
    ],
)

_p4_call = pl.pallas_call(
    _p4_body,
    grid=(_GRID,),
    in_specs=[
        pl.BlockSpec((NC, _BR, F), lambda i: (0, i, 0)),
        pl.BlockSpec((_BR, F), _rows),
        pl.BlockSpec((_BR, 1), _rows),
        pl.BlockSpec((F, 2 * F), _full),
        pl.BlockSpec((1, 2 * F), _full),
        pl.BlockSpec((2 * F, F), _full),
    ],
    out_specs=pl.BlockSpec((_BR, F), _rows),
    out_shape=jax.ShapeDtypeStruct((N, F), jnp.float32),
)

_p6_call = pl.pallas_call(
    _p6_body,
    grid=(_GRID,),
    in_specs=[
        pl.BlockSpec((NC, _BR, F), lambda i: (0, i, 0)),
        pl.BlockSpec((_BR, F), _rows),
        pl.BlockSpec((_BR, 1), _rows),
        pl.BlockSpec((1, F), _full),
    ],
    out_specs=pl.BlockSpec((_BR, F), _rows),
    out_shape=jax.ShapeDtypeStruct((N, F), jnp.float32),
)


def kernel(x, edge_index, W1, b1, W2, b2):
    ei = edge_index.astype(jnp.int32)
    src3 = ei[0].reshape(NW, NBLK, SIB, CH)
    dst3 = ei[1].reshape(NW, NBLK, SIB, CH)
    dst2 = ei[1].reshape(NW, EPW)

    degp = _deg_kernel(dst2)
    degt = degp.T[:N]
    dis, xs = _p2_call(degt, x)
    acc1 = _agg_kernel(src3, dst3, xs)
    gs = _p4_call(acc1, xs, dis, W1, b1.reshape(1, -1), W2)
    acc2 = _agg_kernel(src3, dst3, gs)
    return _p6_call(acc2, gs, dis, b2.reshape(1, -1))


# speedup vs baseline: 1.8080x; 1.0035x over previous
import functools

import jax
import jax.numpy as jnp
from jax import lax
from jax.experimental import pallas as pl
from jax.experimental.pallas import tpu as pltpu
from jax.experimental.pallas import tpu_sc as plsc

N = 10000
NP = 10240
E = 320000
F = 128
NC = 2
NS = 16
NW = NC * NS
CH = 40
NCH = 250
EPW = NCH * CH
SIB = 25
NBLK = NCH // SIB
ND = 5
RPT = NP // NS
NPA = 10112
RPA = NPA // NS

_mesh = plsc.VectorSubcoreMesh(core_axis_name="c", subcore_axis_name="s")


def _zero_vmem(ref, nvec):
    z = jnp.zeros((16,), jnp.float32)

    def body(i, _):
        ref[pl.ds(i * 16, 16)] = z
        return 0

    lax.fori_loop(0, nvec, body, 0)


@functools.partial(
    pl.kernel,
    out_type=jax.ShapeDtypeStruct((NC, NP), jnp.float32),
    mesh=_mesh,
    scratch_types=[
        pltpu.VMEM((EPW,), jnp.int32),
        pltpu.VMEM((NP,), jnp.float32),
        pltpu.VMEM((RPT,), jnp.float32),
        pltpu.VMEM((NS, RPT), jnp.float32),
        pltpu.VMEM_SHARED((NS, NP), jnp.float32),
        pltpu.SemaphoreType.DMA,
    ],
    compiler_params=pltpu.CompilerParams(needs_layout_passes=False),
)
def _deg_kernel(dst_hbm, deg_out, idx_v, deg_v, sum_v, tmp_v, stage_s, sem):
    cid = lax.axis_index("c")
    sid = lax.axis_index("s")
    wid = cid * NS + sid

    pltpu.async_copy(dst_hbm.at[wid], idx_v, sem)
    _zero_vmem(deg_v, NP // 16)
    pltpu.make_async_copy(dst_hbm.at[wid], idx_v, sem).wait()

    ones = jnp.full((16,), 1.0, jnp.float32)

    def body(g, _):
        for u in range(5):
            iv = idx_v[pl.ds((g * 5 + u) * 16, 16)]
            plsc.addupdate_scatter(deg_v, [iv], ones)
        return 0

    lax.fori_loop(0, EPW // 80, body, 0)

    pltpu.sync_copy(deg_v, stage_s.at[sid])
    plsc.subcore_barrier()

    base = sid * RPT
    pltpu.sync_copy(stage_s.at[:, pl.ds(base, RPT)], tmp_v)

    def add(j, _):
        sl = pl.ds(j * 16, 16)
        acc = tmp_v[0, sl]
        for t in range(1, NS):
            acc = acc + tmp_v[t, sl]
        sum_v[sl] = acc
        return 0

    lax.fori_loop(0, RPT // 16, add, 0)
    pltpu.sync_copy(sum_v, deg_out.at[cid, pl.ds(base, RPT)])


@functools.partial(
    pl.kernel,
    out_type=jax.ShapeDtypeStruct((NC, NPA, F), jnp.float32),
    mesh=_mesh,
    scratch_types=[
        pltpu.VMEM((2, SIB, CH), jnp.int32),
        pltpu.VMEM((2, SIB, CH), jnp.int32),
        pltpu.VMEM((CH, F), jnp.float32),
        pltpu.VMEM((CH, F), jnp.float32),
        pltpu.VMEM((CH, F), jnp.float32),
        pltpu.VMEM((CH, F), jnp.float32),
        pltpu.VMEM((CH, F), jnp.float32),
        pltpu.VMEM_SHARED((NPA, F), jnp.float32),
        pltpu.SemaphoreType.DMA,
        pltpu.SemaphoreType.DMA,
        pltpu.SemaphoreType.DMA,
        pltpu.SemaphoreType.DMA,
        pltpu.SemaphoreType.DMA,
        pltpu.SemaphoreType.DMA,
    ],
)
def _agg_kernel(src_hbm, dst_hbm, feat_hbm, acc_out,
                si_v, di_v, rows_0, rows_1, rows_2, rows_3, rows_4, acc_s,
                sem_0, sem_1, sem_2, sem_3, sem_4, sem_i):
    cid = lax.axis_index("c")
    sid = lax.axis_index("s")
    wid = cid * NS + sid

    z = jnp.zeros((16,), jnp.float32)

    def zrow(r, _):
        for j in range(F // 16):
            rows_4[r, pl.ds(j * 16, 16)] = z
        return 0

    lax.fori_loop(0, CH, zrow, 0)

    def init_descs():
        descs = []
        for r in range(RPA // CH):
            descs.append((rows_4, acc_s.at[pl.ds(sid * RPA + r * CH, CH)]))
        rem = RPA % CH
        if rem:
            descs.append((rows_4.at[pl.ds(0, rem)],
                          acc_s.at[pl.ds(sid * RPA + (RPA // CH) * CH, rem)]))
        return descs

    for s, d in init_descs():
        pltpu.async_copy(s, d, sem_4)

    def stage_start(b):
        pltpu.async_copy(src_hbm.at[wid, b], si_v.at[b % 2], sem_i)
        pltpu.async_copy(dst_hbm.at[wid, b], di_v.at[b % 2], sem_i)

    def stage_wait(b):
        pltpu.make_async_copy(src_hbm.at[wid, b], si_v.at[b % 2], sem_i).wait()
        pltpu.make_async_copy(dst_hbm.at[wid, b], di_v.at[b % 2], sem_i).wait()

    def g_start(b, r, buf, sem):
        pltpu.async_copy(feat_hbm.at[si_v.at[b % 2, r]], buf, sem)

    def g_wait(b, r, buf, sem):
        pltpu.make_async_copy(feat_hbm.at[si_v.at[b % 2, r]], buf, sem).wait()

    def s_add(b, r, buf):
        pltpu.sync_copy(buf, acc_s.at[di_v.at[b % 2, r]], add=True)

    bufs = (rows_0, rows_1, rows_2, rows_3, rows_4)
    sems = (sem_0, sem_1, sem_2, sem_3, sem_4)

    stage_start(0)
    stage_wait(0)
    for j in range(ND - 1):
        g_start(0, j, bufs[j], sems[j])
    for s, d in init_descs():
        pltpu.make_async_copy(s, d, sem_4).wait()
    plsc.subcore_barrier()
    g_start(0, ND - 1, bufs[ND - 1], sems[ND - 1])

    for b in range(NBLK):
        if b + 1 < NBLK:
            stage_start(b + 1)
        rot = (SIB * b) % ND
        b4 = tuple(bufs[(rot + j) % ND] for j in range(ND))
        s4 = tuple(sems[(rot + j) % ND] for j in range(ND))

        def body(m, _, b=b, b4=b4, s4=s4):
            l0 = ND * m
            for j in range(ND):
                g_wait(b, l0 + j, b4[j], s4[j])
                s_add(b, l0 + j, b4[j])
                g_start(b, l0 + j + ND, b4[j], s4[j])
            return 0

        nfull = (SIB - ND - (ND - 1) - 1) // ND + 1
        lax.fori_loop(0, nfull, body, 0)

        for l in range(ND * nfull, SIB):
            bf = b4[l % ND]
            sm = s4[l % ND]
            g_wait(b, l, bf, sm)
            s_add(b, l, bf)
            nxt = l + ND
            if nxt < SIB:
                g_start(b, nxt, bf, sm)
            elif b + 1 < NBLK:
                if nxt == SIB:
                    stage_wait(b + 1)
                g_start(b + 1, nxt - SIB, bf, sm)

    plsc.subcore_barrier()
    pltpu.sync_copy(acc_s.at[pl.ds(sid * RPA, RPA)],
                    acc_out.at[cid, pl.ds(sid * RPA, RPA)])


_GRID = 10
_BR = N // _GRID


def _p2_body(degt_ref, x_ref, dis_ref, xs_ref):
    deg = degt_ref[:, 0:1] + degt_ref[:, 1:2] + 1.0
    dis = lax.rsqrt(deg)
    dis_ref[...] = dis
    xs_ref[...] = dis * x_ref[...]


def _p4_body(acc_ref, xs_ref, dis_ref, w1_ref, b1_ref, w2_ref, gs_ref):
    z1 = dis_ref[...] * (acc_ref[0] + acc_ref[1] + xs_ref[...])
    h = jnp.dot(z1, w1_ref[...], preferred_element_type=jnp.float32)
    h = jnp.maximum(h + b1_ref[...], 0.0)
    g = jnp.dot(h, w2_ref[...], preferred_element_type=jnp.float32)
    gs_ref[...] = dis_ref[...] * g


def _p6_body(acc_ref, gs_ref, dis_ref, b2_ref, out_ref):
    out_ref[...] = (dis_ref[...] * (acc_ref[0] + acc_ref[1] + gs_ref[...])
                    + b2_ref[...])


def _rows(i):
    return (i, 0)


def _full(i):
    return (0, 0)


_p2_call = pl.pallas_call(
    _p2_body,
    grid=(_GRID,),
    in_specs=[
        pl.BlockSpec((_BR, 2), _rows),
        pl.BlockSpec((_BR, F), _rows),
    ],
    out_specs=[
        pl.BlockSpec((_BR, 1), _rows),
        pl.BlockSpec((_BR, F), _rows),
    ],
    out_shape=[
        jax.ShapeDtypeStruct((N, 1), jnp.float32),
        jax.ShapeDtypeStruct((N, F), jnp.float32),
    ],
)

_p4_call = pl.pallas_call(
    _p4_body,
    grid=(_GRID,),
    in_specs=[
        pl.BlockSpec((NC, _BR, F), lambda i: (0, i, 0)),
        pl.BlockSpec((_BR, F), _rows),
        pl.BlockSpec((_BR, 1), _rows),
        pl.BlockSpec((F, 2 * F), _full),
        pl.BlockSpec((1, 2 * F), _full),
        pl.BlockSpec((2 * F, F), _full),
    ],
    out_specs=pl.BlockSpec((_BR, F), _rows),
    out_shape=jax.ShapeDtypeStruct((N, F), jnp.float32),
)

_p6_call = pl.pallas_call(
    _p6_body,
    grid=(_GRID,),
    in_specs=[
        pl.BlockSpec((NC, _BR, F), lambda i: (0, i, 0)),
        pl.BlockSpec((_BR, F), _rows),
        pl.BlockSpec((_BR, 1), _rows),
        pl.BlockSpec((1, F), _full),
    ],
    out_specs=pl.BlockSpec((_BR, F), _rows),
    out_shape=jax.ShapeDtypeStruct((N, F), jnp.float32),
)


def kernel(x, edge_index, W1, b1, W2, b2):
    ei = edge_index.astype(jnp.int32)
    src3 = ei[0].reshape(NW, NBLK, SIB, CH)
    dst3 = ei[1].reshape(NW, NBLK, SIB, CH)
    dst2 = ei[1].reshape(NW, EPW)

    degp = _deg_kernel(dst2)
    degt = degp.T[:N]
    dis, xs = _p2_call(degt, x)
    acc1 = _agg_kernel(src3, dst3, xs)
    gs = _p4_call(acc1, xs, dis, W1, b1.reshape(1, -1), W2)
    acc2 = _agg_kernel(src3, dst3, gs)
    return _p6_call(acc2, gs, dis, b2.reshape(1, -1))
